# baseline (speedup 1.0000x reference)
import functools

import jax
import jax.numpy as jnp
from jax import lax
from jax.experimental import pallas as pl
from jax.experimental.pallas import tpu as pltpu
from jax.experimental.pallas import tpu_sc as plsc

NC = 2
NS = 16
L = 16
CH = 128
SLAB = 8


def _spmm_body(npad, nrnd, s1, s2, s3, e1, e2, e3, part,
               acc, gi, rows, gsem):
    c = lax.axis_index("c")
    s = lax.axis_index("s")
    w = c * NS + s
    rpt = npad // NS

    @pl.loop(0, CH)
    def _zero_rows(i):
        for j in range(rows.shape[1] // L):
            rows[i, j * L:(j + 1) * L] = jnp.zeros((L,), jnp.float32)

    for k in range(rpt // CH):
        pltpu.sync_copy(rows, acc.at[pl.ds(s * rpt + k * CH, CH)])
    plsc.subcore_barrier()

    streams = ((e1, s1), (e2, s2), (e3, s3))

    nslabt = nrnd // SLAB
    assert nrnd % SLAB == 0

    for e_ref, s_ref in streams:
        base = w * nslabt

        @pl.loop(0, nslabt)
        def _slab(m, _e=e_ref, _s=s_ref):
            pltpu.sync_copy(_e.at[0, base + m], gi.at[0])
            pltpu.sync_copy(_e.at[1, base + m], gi.at[1])
            for b in range(SLAB):
                for d in (0, 1):
                    pltpu.async_copy(_s.at[gi.at[d, b]], rows, gsem).wait()
                    pltpu.sync_copy(rows, acc.at[gi.at[1 - d, b]], add=True)

    plsc.subcore_barrier()
    pltpu.sync_copy(acc.at[pl.ds(s * rpt, rpt)],
                    part.at[c, pl.ds(s * rpt, rpt)])


def _sc_spmm(s1, s2, s3, e1, e2, e3):
    npad, d_model = s1.shape
    nchunk = e1.shape[1] * SLAB
    nrnd = nchunk // (NC * NS)
    mesh = plsc.VectorSubcoreMesh(core_axis_name="c", subcore_axis_name="s")
    body = functools.partial(_spmm_body, npad, nrnd)
    return pl.kernel(
        body,
        out_type=jax.ShapeDtypeStruct((NC, npad, d_model), jnp.float32),
        mesh=mesh,
        scratch_types=[
            pltpu.VMEM_SHARED((npad, d_model), jnp.float32),
            pltpu.VMEM((2, SLAB, CH), jnp.int32),
            pltpu.VMEM((CH, d_model), jnp.float32),
            pltpu.SemaphoreType.DMA,
        ],
    )(s1, s2, s3, e1, e2, e3)


def _tc_scaled_support_body(x_ref, w_ref, wb_ref, s1_ref, s2_ref, s3_ref):
    sup = jnp.dot(x_ref[...], w_ref[...], preferred_element_type=jnp.float32)
    s1_ref[...] = wb_ref[0, 0] * sup
    s2_ref[...] = wb_ref[1, 0] * sup
    s3_ref[...] = wb_ref[2, 0] * sup


def _tc_scaled_support(x, w, wb, bm):
    n, d_model = x.shape
    grid = (n // bm,)
    blk = pl.BlockSpec((bm, d_model), lambda i: (i, 0))
    return pl.pallas_call(
        _tc_scaled_support_body,
        grid=grid,
        in_specs=[blk,
                  pl.BlockSpec((d_model, d_model), lambda i: (0, 0)),
                  pl.BlockSpec(memory_space=pltpu.SMEM)],
        out_specs=[blk, blk, blk],
        out_shape=[jax.ShapeDtypeStruct((n, d_model), jnp.float32)] * 3,
    )(x, w, wb)


def _tc_merge_support_body(p_ref, b_ref, w_ref, wb_ref,
                           u_ref, s1_ref, s2_ref, s3_ref):
    u = p_ref[0] + p_ref[1] + b_ref[...]
    u_ref[...] = u
    sup = jnp.dot(u, w_ref[...], preferred_element_type=jnp.float32)
    s1_ref[...] = wb_ref[0, 0] * sup
    s2_ref[...] = wb_ref[1, 0] * sup
    s3_ref[...] = wb_ref[2, 0] * sup


def _tc_merge_support(p, b, w, wb, bm):
    _, n, d_model = p.shape
    grid = (n // bm,)
    blk = pl.BlockSpec((bm, d_model), lambda i: (i, 0))
    return pl.pallas_call(
        _tc_merge_support_body,
        grid=grid,
        in_specs=[pl.BlockSpec((2, bm, d_model), lambda i: (0, i, 0)),
                  pl.BlockSpec((1, d_model), lambda i: (0, 0)),
                  pl.BlockSpec((d_model, d_model), lambda i: (0, 0)),
                  pl.BlockSpec(memory_space=pltpu.SMEM)],
        out_specs=[blk, blk, blk, blk],
        out_shape=[jax.ShapeDtypeStruct((n, d_model), jnp.float32)] * 4,
    )(p, b.reshape(1, d_model), w, wb)


def _tc_final_body(u1_ref, q_ref, b_ref, out_ref):
    out_ref[...] = 0.5 * (u1_ref[...] + q_ref[0] + q_ref[1] + b_ref[...])


def _tc_final(u1, q, b, bm, n):
    _, d_model = u1.shape
    grid = (n // bm,)
    blk = pl.BlockSpec((bm, d_model), lambda i: (i, 0))
    return pl.pallas_call(
        _tc_final_body,
        grid=grid,
        in_specs=[blk,
                  pl.BlockSpec((2, bm, d_model), lambda i: (0, i, 0)),
                  pl.BlockSpec((1, d_model), lambda i: (0, 0))],
        out_specs=blk,
        out_shape=jax.ShapeDtypeStruct((n, d_model), jnp.float32),
    )(u1, q, b.reshape(1, d_model))


def _pad_edges(e, n, nchunk_pad):
    e = e.astype(jnp.int32)
    pad = nchunk_pad * CH - e.shape[1]
    ep = jnp.pad(e, ((0, 0), (0, pad)), constant_values=n)
    return ep.reshape(2, nchunk_pad // SLAB, SLAB, CH)


def kernel(x, edge_index1, edge_index2, edge_index3, weight_b, W1, b1, W2, b2):
    n, d_model = x.shape
    npad = ((n + NS * CH - 1) // (NS * CH)) * (NS * CH)
    e = edge_index1.shape[1]
    echunk = NC * NS * SLAB * CH
    nchunk_pad = ((e + echunk - 1) // echunk) * echunk // CH
    e1 = _pad_edges(edge_index1, n, nchunk_pad)
    e2 = _pad_edges(edge_index2, n, nchunk_pad)
    e3 = _pad_edges(edge_index3, n, nchunk_pad)
    x_pad = jnp.pad(x, ((0, npad - n), (0, 0)))

    s1, s2, s3 = _tc_scaled_support(x_pad, W1, weight_b, npad // 16)
    p = _sc_spmm(s1, s2, s3, e1, e2, e3)
    u1, t1, t2, t3 = _tc_merge_support(p, b1, W2, weight_b, npad // 16)
    q = _sc_spmm(t1, t2, t3, e1, e2, e3)
    return _tc_final(u1, q, b2, n // 10, n)


# speedup vs baseline: 2.8961x; 2.8961x over previous
import functools

import jax
import jax.numpy as jnp
from jax import lax
from jax.experimental import pallas as pl
from jax.experimental.pallas import tpu as pltpu
from jax.experimental.pallas import tpu_sc as plsc

NC = 2
NS = 16
L = 16
CH = 128
SLAB = 8


def _spmm_body(npad, nrnd, s1, s2, s3, e1, e2, e3, part,
               acc, gi, rows, gsem):
    c = lax.axis_index("c")
    s = lax.axis_index("s")
    w = c * NS + s
    rpt = npad // NS

    @pl.loop(0, CH)
    def _zero_rows(i):
        for j in range(rows.shape[1] // L):
            rows[i, j * L:(j + 1) * L] = jnp.zeros((L,), jnp.float32)

    for k in range(rpt // CH):
        pltpu.sync_copy(rows, acc.at[pl.ds(s * rpt + k * CH, CH)])
    plsc.subcore_barrier()

    streams = ((e1, s1), (e2, s2), (e3, s3))

    nslabt = nrnd // SLAB
    assert nrnd % SLAB == 0

    for e_ref, s_ref in streams:
        base = w * nslabt

        @pl.loop(0, nslabt)
        def _slab(m, _e=e_ref, _s=s_ref):
            pltpu.sync_copy(_e.at[0, base + m], gi.at[0])
            pltpu.sync_copy(_e.at[1, base + m], gi.at[1])
            for b in range(SLAB):
                for d in (0, 1):
                    pltpu.async_copy(_s.at[gi.at[d, b]], rows, gsem).wait()
                    pltpu.sync_copy(rows, acc.at[gi.at[1 - d, b]], add=True)

    plsc.subcore_barrier()
    pltpu.sync_copy(acc.at[pl.ds(s * rpt, rpt)],
                    part.at[c, pl.ds(s * rpt, rpt)])


def _sc_spmm(s1, s2, s3, e1, e2, e3):
    npad, d_model = s1.shape
    nchunk = e1.shape[1] * SLAB
    nrnd = nchunk // (NC * NS)
    mesh = plsc.VectorSubcoreMesh(core_axis_name="c", subcore_axis_name="s")
    body = functools.partial(_spmm_body, npad, nrnd)
    return pl.kernel(
        body,
        out_type=jax.ShapeDtypeStruct((NC, npad, d_model), jnp.float32),
        mesh=mesh,
        scratch_types=[
            pltpu.VMEM_SHARED((npad, d_model), jnp.float32),
            pltpu.VMEM((2, SLAB, CH), jnp.int32),
            pltpu.VMEM((CH, d_model), jnp.float32),
            pltpu.SemaphoreType.DMA,
        ],
    )(s1, s2, s3, e1, e2, e3)


def _tc_scaled_support_body(x_ref, w_ref, wb_ref, s1_ref, s2_ref, s3_ref):
    sup = jnp.dot(x_ref[...], w_ref[...], preferred_element_type=jnp.float32)
    s1_ref[...] = wb_ref[0, 0] * sup
    s2_ref[...] = wb_ref[1, 0] * sup
    s3_ref[...] = wb_ref[2, 0] * sup


def _tc_scaled_support(x, w, wb, bm):
    n, d_model = x.shape
    grid = (n // bm,)
    blk = pl.BlockSpec((bm, d_model), lambda i: (i, 0))
    return pl.pallas_call(
        _tc_scaled_support_body,
        grid=grid,
        in_specs=[blk,
                  pl.BlockSpec((d_model, d_model), lambda i: (0, 0)),
                  pl.BlockSpec(memory_space=pltpu.SMEM)],
        out_specs=[blk, blk, blk],
        out_shape=[jax.ShapeDtypeStruct((n, d_model), jnp.float32)] * 3,
    )(x, w, wb)


def _tc_merge_support_body(p_ref, b_ref, w_ref, wb_ref,
                           u_ref, s1_ref, s2_ref, s3_ref):
    u = p_ref[0] + p_ref[1] + b_ref[...]
    u_ref[...] = u
    sup = jnp.dot(u, w_ref[...], preferred_element_type=jnp.float32)
    s1_ref[...] = wb_ref[0, 0] * sup
    s2_ref[...] = wb_ref[1, 0] * sup
    s3_ref[...] = wb_ref[2, 0] * sup


def _tc_merge_support(p, b, w, wb, bm):
    _, n, d_model = p.shape
    grid = (n // bm,)
    blk = pl.BlockSpec((bm, d_model), lambda i: (i, 0))
    return pl.pallas_call(
        _tc_merge_support_body,
        grid=grid,
        in_specs=[pl.BlockSpec((2, bm, d_model), lambda i: (0, i, 0)),
                  pl.BlockSpec((1, d_model), lambda i: (0, 0)),
                  pl.BlockSpec((d_model, d_model), lambda i: (0, 0)),
                  pl.BlockSpec(memory_space=pltpu.SMEM)],
        out_specs=[blk, blk, blk, blk],
        out_shape=[jax.ShapeDtypeStruct((n, d_model), jnp.float32)] * 4,
    )(p, b.reshape(1, d_model), w, wb)


def _tc_final_body(u1_ref, q_ref, b_ref, out_ref):
    out_ref[...] = 0.5 * (u1_ref[...] + q_ref[0] + q_ref[1] + b_ref[...])


def _tc_final(u1, q, b, bm, n):
    _, d_model = u1.shape
    grid = (n // bm,)
    blk = pl.BlockSpec((bm, d_model), lambda i: (i, 0))
    return pl.pallas_call(
        _tc_final_body,
        grid=grid,
        in_specs=[blk,
                  pl.BlockSpec((2, bm, d_model), lambda i: (0, i, 0)),
                  pl.BlockSpec((1, d_model), lambda i: (0, 0))],
        out_specs=blk,
        out_shape=jax.ShapeDtypeStruct((n, d_model), jnp.float32),
    )(u1, q, b.reshape(1, d_model))


def _pad_edges(e, n, npad, nchunk_pad):
    e = e.astype(jnp.int32)
    pad = nchunk_pad * CH - e.shape[1]
    fill = n + (jnp.arange(pad, dtype=jnp.int32) % (npad - n))
    ep = jnp.concatenate(
        [e, jnp.broadcast_to(fill, (2, pad))], axis=1)
    return ep.reshape(2, nchunk_pad // SLAB, SLAB, CH)


def kernel(x, edge_index1, edge_index2, edge_index3, weight_b, W1, b1, W2, b2):
    n, d_model = x.shape
    npad = ((n + NS * CH - 1) // (NS * CH)) * (NS * CH)
    e = edge_index1.shape[1]
    echunk = NC * NS * SLAB * CH
    nchunk_pad = ((e + echunk - 1) // echunk) * echunk // CH
    e1 = _pad_edges(edge_index1, n, npad, nchunk_pad)
    e2 = _pad_edges(edge_index2, n, npad, nchunk_pad)
    e3 = _pad_edges(edge_index3, n, npad, nchunk_pad)
    x_pad = jnp.pad(x, ((0, npad - n), (0, 0)))

    s1, s2, s3 = _tc_scaled_support(x_pad, W1, weight_b, npad // 16)
    p = _sc_spmm(s1, s2, s3, e1, e2, e3)
    u1, t1, t2, t3 = _tc_merge_support(p, b1, W2, weight_b, npad // 16)
    q = _sc_spmm(t1, t2, t3, e1, e2, e3)
    return _tc_final(u1, q, b2, n // 10, n)


# speedup vs baseline: 3.8237x; 1.3203x over previous
import functools

import jax
import jax.numpy as jnp
from jax import lax
from jax.experimental import pallas as pl
from jax.experimental.pallas import tpu as pltpu
from jax.experimental.pallas import tpu_sc as plsc

NC = 2
NS = 16
L = 16
CH = 128
SLAB = 8


def _spmm_body(npad, nrnd, s1, s2, s3, e1, e2, e3, part,
               acc, gi, rows, gsem):
    c = lax.axis_index("c")
    s = lax.axis_index("s")
    w = c * NS + s
    rpt = npad // NS

    @pl.loop(0, CH)
    def _zero_rows(i):
        for j in range(rows.shape[2] // L):
            rows[0, i, j * L:(j + 1) * L] = jnp.zeros((L,), jnp.float32)

    for k in range(rpt // CH):
        pltpu.sync_copy(rows.at[0], acc.at[pl.ds(s * rpt + k * CH, CH)])
    plsc.subcore_barrier()

    streams = ((e1, s1), (e2, s2), (e3, s3))

    nslabt = nrnd // SLAB
    assert nrnd % SLAB == 0

    for e_ref, s_ref in streams:
        base = w * nslabt

        @pl.loop(0, nslabt)
        def _slab(m, _e=e_ref, _s=s_ref):
            pltpu.sync_copy(_e.at[0, base + m], gi.at[0])
            pltpu.sync_copy(_e.at[1, base + m], gi.at[1])
            pltpu.async_copy(_s.at[gi.at[0, 0]], rows.at[0], gsem.at[0])
            for u in range(2 * SLAB):
                b, d = divmod(u, 2)
                rb = u % 2
                pltpu.make_async_copy(_s.at[gi.at[d, b]], rows.at[rb],
                                      gsem.at[rb]).wait()
                if u + 1 < 2 * SLAB:
                    b2, d2 = divmod(u + 1, 2)
                    pltpu.async_copy(_s.at[gi.at[d2, b2]], rows.at[1 - rb],
                                     gsem.at[1 - rb])
                pltpu.sync_copy(rows.at[rb], acc.at[gi.at[1 - d, b]],
                                add=True)

    plsc.subcore_barrier()
    pltpu.sync_copy(acc.at[pl.ds(s * rpt, rpt)],
                    part.at[c, pl.ds(s * rpt, rpt)])


def _sc_spmm(s1, s2, s3, e1, e2, e3):
    npad, d_model = s1.shape
    nchunk = e1.shape[1] * SLAB
    nrnd = nchunk // (NC * NS)
    mesh = plsc.VectorSubcoreMesh(core_axis_name="c", subcore_axis_name="s")
    body = functools.partial(_spmm_body, npad, nrnd)
    return pl.kernel(
        body,
        out_type=jax.ShapeDtypeStruct((NC, npad, d_model), jnp.float32),
        mesh=mesh,
        scratch_types=[
            pltpu.VMEM_SHARED((npad, d_model), jnp.float32),
            pltpu.VMEM((2, SLAB, CH), jnp.int32),
            pltpu.VMEM((2, CH, d_model), jnp.float32),
            pltpu.SemaphoreType.DMA((2,)),
        ],
    )(s1, s2, s3, e1, e2, e3)


def _tc_scaled_support_body(x_ref, w_ref, wb_ref, s1_ref, s2_ref, s3_ref):
    sup = jnp.dot(x_ref[...], w_ref[...], preferred_element_type=jnp.float32)
    s1_ref[...] = wb_ref[0, 0] * sup
    s2_ref[...] = wb_ref[1, 0] * sup
    s3_ref[...] = wb_ref[2, 0] * sup


def _tc_scaled_support(x, w, wb, bm):
    n, d_model = x.shape
    grid = (n // bm,)
    blk = pl.BlockSpec((bm, d_model), lambda i: (i, 0))
    return pl.pallas_call(
        _tc_scaled_support_body,
        grid=grid,
        in_specs=[blk,
                  pl.BlockSpec((d_model, d_model), lambda i: (0, 0)),
                  pl.BlockSpec(memory_space=pltpu.SMEM)],
        out_specs=[blk, blk, blk],
        out_shape=[jax.ShapeDtypeStruct((n, d_model), jnp.float32)] * 3,
    )(x, w, wb)


def _tc_merge_support_body(p_ref, b_ref, w_ref, wb_ref,
                           u_ref, s1_ref, s2_ref, s3_ref):
    u = p_ref[0] + p_ref[1] + b_ref[...]
    u_ref[...] = u
    sup = jnp.dot(u, w_ref[...], preferred_element_type=jnp.float32)
    s1_ref[...] = wb_ref[0, 0] * sup
    s2_ref[...] = wb_ref[1, 0] * sup
    s3_ref[...] = wb_ref[2, 0] * sup


def _tc_merge_support(p, b, w, wb, bm):
    _, n, d_model = p.shape
    grid = (n // bm,)
    blk = pl.BlockSpec((bm, d_model), lambda i: (i, 0))
    return pl.pallas_call(
        _tc_merge_support_body,
        grid=grid,
        in_specs=[pl.BlockSpec((2, bm, d_model), lambda i: (0, i, 0)),
                  pl.BlockSpec((1, d_model), lambda i: (0, 0)),
                  pl.BlockSpec((d_model, d_model), lambda i: (0, 0)),
                  pl.BlockSpec(memory_space=pltpu.SMEM)],
        out_specs=[blk, blk, blk, blk],
        out_shape=[jax.ShapeDtypeStruct((n, d_model), jnp.float32)] * 4,
    )(p, b.reshape(1, d_model), w, wb)


def _tc_final_body(u1_ref, q_ref, b_ref, out_ref):
    out_ref[...] = 0.5 * (u1_ref[...] + q_ref[0] + q_ref[1] + b_ref[...])


def _tc_final(u1, q, b, bm, n):
    _, d_model = u1.shape
    grid = (n // bm,)
    blk = pl.BlockSpec((bm, d_model), lambda i: (i, 0))
    return pl.pallas_call(
        _tc_final_body,
        grid=grid,
        in_specs=[blk,
                  pl.BlockSpec((2, bm, d_model), lambda i: (0, i, 0)),
                  pl.BlockSpec((1, d_model), lambda i: (0, 0))],
        out_specs=blk,
        out_shape=jax.ShapeDtypeStruct((n, d_model), jnp.float32),
    )(u1, q, b.reshape(1, d_model))


def _pad_edges(e, n, npad, nchunk_pad):
    e = e.astype(jnp.int32)
    pad = nchunk_pad * CH - e.shape[1]
    fill = n + (jnp.arange(pad, dtype=jnp.int32) % (npad - n))
    ep = jnp.concatenate(
        [e, jnp.broadcast_to(fill, (2, pad))], axis=1)
    return ep.reshape(2, nchunk_pad // SLAB, SLAB, CH)


def kernel(x, edge_index1, edge_index2, edge_index3, weight_b, W1, b1, W2, b2):
    n, d_model = x.shape
    npad = ((n + NS * CH - 1) // (NS * CH)) * (NS * CH)
    e = edge_index1.shape[1]
    echunk = NC * NS * SLAB * CH
    nchunk_pad = ((e + echunk - 1) // echunk) * echunk // CH
    e1 = _pad_edges(edge_index1, n, npad, nchunk_pad)
    e2 = _pad_edges(edge_index2, n, npad, nchunk_pad)
    e3 = _pad_edges(edge_index3, n, npad, nchunk_pad)
    x_pad = jnp.pad(x, ((0, npad - n), (0, 0)))

    s1, s2, s3 = _tc_scaled_support(x_pad, W1, weight_b, npad // 16)
    p = _sc_spmm(s1, s2, s3, e1, e2, e3)
    u1, t1, t2, t3 = _tc_merge_support(p, b1, W2, weight_b, npad // 16)
    q = _sc_spmm(t1, t2, t3, e1, e2, e3)
    return _tc_final(u1, q, b2, n // 10, n)


# speedup vs baseline: 4.0250x; 1.0527x over previous
import functools

import jax
import jax.numpy as jnp
from jax import lax
from jax.experimental import pallas as pl
from jax.experimental.pallas import tpu as pltpu
from jax.experimental.pallas import tpu_sc as plsc

NC = 2
NS = 16
L = 16
CH = 128
SLAB = 8


def _spmm_body(npad, nrnd, s1, s2, s3, e1, e2, e3, part,
               acc, gi, rows, gsem):
    c = lax.axis_index("c")
    s = lax.axis_index("s")
    w = c * NS + s
    rpt = npad // NS

    @pl.loop(0, CH)
    def _zero_rows(i):
        for j in range(rows.shape[2] // L):
            rows[0, i, j * L:(j + 1) * L] = jnp.zeros((L,), jnp.float32)

    for k in range(rpt // CH):
        pltpu.sync_copy(rows.at[0], acc.at[pl.ds(s * rpt + k * CH, CH)])
    plsc.subcore_barrier()

    streams = ((e1, s1), (e2, s2), (e3, s3))

    nslabt = nrnd // SLAB
    assert nrnd % SLAB == 0

    for e_ref, s_ref in streams:
        base = w * nslabt

        pltpu.sync_copy(e_ref.at[0, pl.ds(base, nslabt)], gi.at[0])
        pltpu.sync_copy(e_ref.at[1, pl.ds(base, nslabt)], gi.at[1])
        pltpu.async_copy(s_ref.at[gi.at[0, 0, 0]], rows.at[0], gsem.at[0])

        @pl.loop(0, nslabt)
        def _slab(m, _s=s_ref):
            for u in range(2 * SLAB):
                b, d = divmod(u, 2)
                rb = u % 2
                pltpu.make_async_copy(_s.at[gi.at[d, m, b]], rows.at[rb],
                                      gsem.at[rb]).wait()
                if u + 1 < 2 * SLAB:
                    b2, d2 = divmod(u + 1, 2)
                    pltpu.async_copy(_s.at[gi.at[d2, m, b2]],
                                     rows.at[1 - rb], gsem.at[1 - rb])
                else:
                    @pl.when(m + 1 < nslabt)
                    def _():
                        pltpu.async_copy(_s.at[gi.at[0, m + 1, 0]],
                                         rows.at[1 - rb], gsem.at[1 - rb])
                pltpu.sync_copy(rows.at[rb], acc.at[gi.at[1 - d, m, b]],
                                add=True)

    plsc.subcore_barrier()
    pltpu.sync_copy(acc.at[pl.ds(s * rpt, rpt)],
                    part.at[c, pl.ds(s * rpt, rpt)])


def _sc_spmm(s1, s2, s3, e1, e2, e3):
    npad, d_model = s1.shape
    nchunk = e1.shape[1] * SLAB
    nrnd = nchunk // (NC * NS)
    mesh = plsc.VectorSubcoreMesh(core_axis_name="c", subcore_axis_name="s")
    body = functools.partial(_spmm_body, npad, nrnd)
    return pl.kernel(
        body,
        out_type=jax.ShapeDtypeStruct((NC, npad, d_model), jnp.float32),
        mesh=mesh,
        scratch_types=[
            pltpu.VMEM_SHARED((npad, d_model), jnp.float32),
            pltpu.VMEM((2, nrnd // SLAB, SLAB, CH), jnp.int32),
            pltpu.VMEM((2, CH, d_model), jnp.float32),
            pltpu.SemaphoreType.DMA((2,)),
        ],
    )(s1, s2, s3, e1, e2, e3)


def _tc_scaled_support_body(x_ref, w_ref, wb_ref, s1_ref, s2_ref, s3_ref):
    sup = jnp.dot(x_ref[...], w_ref[...], preferred_element_type=jnp.float32)
    s1_ref[...] = wb_ref[0, 0] * sup
    s2_ref[...] = wb_ref[1, 0] * sup
    s3_ref[...] = wb_ref[2, 0] * sup


def _tc_scaled_support(x, w, wb, bm):
    n, d_model = x.shape
    grid = (n // bm,)
    blk = pl.BlockSpec((bm, d_model), lambda i: (i, 0))
    return pl.pallas_call(
        _tc_scaled_support_body,
        grid=grid,
        in_specs=[blk,
                  pl.BlockSpec((d_model, d_model), lambda i: (0, 0)),
                  pl.BlockSpec(memory_space=pltpu.SMEM)],
        out_specs=[blk, blk, blk],
        out_shape=[jax.ShapeDtypeStruct((n, d_model), jnp.float32)] * 3,
    )(x, w, wb)


def _tc_merge_support_body(p_ref, b_ref, w_ref, wb_ref,
                           u_ref, s1_ref, s2_ref, s3_ref):
    u = p_ref[0] + p_ref[1] + b_ref[...]
    u_ref[...] = u
    sup = jnp.dot(u, w_ref[...], preferred_element_type=jnp.float32)
    s1_ref[...] = wb_ref[0, 0] * sup
    s2_ref[...] = wb_ref[1, 0] * sup
    s3_ref[...] = wb_ref[2, 0] * sup


def _tc_merge_support(p, b, w, wb, bm):
    _, n, d_model = p.shape
    grid = (n // bm,)
    blk = pl.BlockSpec((bm, d_model), lambda i: (i, 0))
    return pl.pallas_call(
        _tc_merge_support_body,
        grid=grid,
        in_specs=[pl.BlockSpec((2, bm, d_model), lambda i: (0, i, 0)),
                  pl.BlockSpec((1, d_model), lambda i: (0, 0)),
                  pl.BlockSpec((d_model, d_model), lambda i: (0, 0)),
                  pl.BlockSpec(memory_space=pltpu.SMEM)],
        out_specs=[blk, blk, blk, blk],
        out_shape=[jax.ShapeDtypeStruct((n, d_model), jnp.float32)] * 4,
    )(p, b.reshape(1, d_model), w, wb)


def _tc_final_body(u1_ref, q_ref, b_ref, out_ref):
    out_ref[...] = 0.5 * (u1_ref[...] + q_ref[0] + q_ref[1] + b_ref[...])


def _tc_final(u1, q, b, bm, n):
    _, d_model = u1.shape
    grid = (n // bm,)
    blk = pl.BlockSpec((bm, d_model), lambda i: (i, 0))
    return pl.pallas_call(
        _tc_final_body,
        grid=grid,
        in_specs=[blk,
                  pl.BlockSpec((2, bm, d_model), lambda i: (0, i, 0)),
                  pl.BlockSpec((1, d_model), lambda i: (0, 0))],
        out_specs=blk,
        out_shape=jax.ShapeDtypeStruct((n, d_model), jnp.float32),
    )(u1, q, b.reshape(1, d_model))


def _pad_edges(e, n, npad, nchunk_pad):
    e = e.astype(jnp.int32)
    pad = nchunk_pad * CH - e.shape[1]
    fill = n + (jnp.arange(pad, dtype=jnp.int32) % (npad - n))
    ep = jnp.concatenate(
        [e, jnp.broadcast_to(fill, (2, pad))], axis=1)
    return ep.reshape(2, nchunk_pad // SLAB, SLAB, CH)


def kernel(x, edge_index1, edge_index2, edge_index3, weight_b, W1, b1, W2, b2):
    n, d_model = x.shape
    npad = ((n + NS * CH - 1) // (NS * CH)) * (NS * CH)
    e = edge_index1.shape[1]
    echunk = NC * NS * SLAB * CH
    nchunk_pad = ((e + echunk - 1) // echunk) * echunk // CH
    e1 = _pad_edges(edge_index1, n, npad, nchunk_pad)
    e2 = _pad_edges(edge_index2, n, npad, nchunk_pad)
    e3 = _pad_edges(edge_index3, n, npad, nchunk_pad)
    x_pad = jnp.pad(x, ((0, npad - n), (0, 0)))

    s1, s2, s3 = _tc_scaled_support(x_pad, W1, weight_b, npad // 16)
    p = _sc_spmm(s1, s2, s3, e1, e2, e3)
    u1, t1, t2, t3 = _tc_merge_support(p, b1, W2, weight_b, npad // 16)
    q = _sc_spmm(t1, t2, t3, e1, e2, e3)
    return _tc_final(u1, q, b2, n // 10, n)


# speedup vs baseline: 4.0321x; 1.0018x over previous
import functools

import jax
import jax.numpy as jnp
from jax import lax
from jax.experimental import pallas as pl
from jax.experimental.pallas import tpu as pltpu
from jax.experimental.pallas import tpu_sc as plsc

NC = 2
NS = 16
L = 16
CH = 128
SLAB = 8


def _spmm_body(npad, nrnd, s1, s2, s3, e1, e2, e3, part,
               acc, gi, rows, gsem, ssem):
    c = lax.axis_index("c")
    s = lax.axis_index("s")
    w = c * NS + s
    rpt = npad // NS

    @pl.loop(0, CH)
    def _zero_rows(i):
        for j in range(rows.shape[2] // L):
            rows[0, i, j * L:(j + 1) * L] = jnp.zeros((L,), jnp.float32)

    for k in range(rpt // CH):
        pltpu.sync_copy(rows.at[0], acc.at[pl.ds(s * rpt + k * CH, CH)])
    plsc.subcore_barrier()

    streams = ((e1, s1), (e2, s2), (e3, s3))

    nslabt = nrnd // SLAB
    assert nrnd % SLAB == 0

    for e_ref, s_ref in streams:
        base = w * nslabt

        pltpu.sync_copy(e_ref.at[0, pl.ds(base, nslabt)], gi.at[0])
        pltpu.sync_copy(e_ref.at[1, pl.ds(base, nslabt)], gi.at[1])
        pltpu.async_copy(s_ref.at[gi.at[0, 0, 0]], rows.at[0], gsem.at[0])

        @pl.loop(0, nslabt)
        def _slab(m, _s=s_ref):
            for u in range(2 * SLAB):
                b, d = divmod(u, 2)
                rb = u % 2

                pltpu.make_async_copy(_s.at[gi.at[d, m, b]], rows.at[rb],
                                      gsem.at[rb]).wait()

                def _wait_prev_scatter(_rb=1 - rb):
                    pltpu.make_async_copy(rows.at[_rb], acc.at[gi.at[0, m, 0]],
                                          ssem.at[_rb]).wait()

                if u == 0:
                    @pl.when(m >= 1)
                    def _():
                        _wait_prev_scatter()
                else:
                    _wait_prev_scatter()

                if u + 1 < 2 * SLAB:
                    b2, d2 = divmod(u + 1, 2)
                    pltpu.async_copy(_s.at[gi.at[d2, m, b2]],
                                     rows.at[1 - rb], gsem.at[1 - rb])
                else:
                    @pl.when(m + 1 < nslabt)
                    def _():
                        pltpu.async_copy(_s.at[gi.at[0, m + 1, 0]],
                                         rows.at[1 - rb], gsem.at[1 - rb])
                pltpu.async_copy(rows.at[rb], acc.at[gi.at[1 - d, m, b]],
                                 ssem.at[rb], add=True)

        pltpu.make_async_copy(rows.at[1], acc.at[gi.at[0, 0, 0]],
                              ssem.at[1]).wait()

    plsc.subcore_barrier()
    pltpu.sync_copy(acc.at[pl.ds(s * rpt, rpt)],
                    part.at[c, pl.ds(s * rpt, rpt)])


def _sc_spmm(s1, s2, s3, e1, e2, e3):
    npad, d_model = s1.shape
    nchunk = e1.shape[1] * SLAB
    nrnd = nchunk // (NC * NS)
    mesh = plsc.VectorSubcoreMesh(core_axis_name="c", subcore_axis_name="s")
    body = functools.partial(_spmm_body, npad, nrnd)
    return pl.kernel(
        body,
        out_type=jax.ShapeDtypeStruct((NC, npad, d_model), jnp.float32),
        mesh=mesh,
        scratch_types=[
            pltpu.VMEM_SHARED((npad, d_model), jnp.float32),
            pltpu.VMEM((2, nrnd // SLAB, SLAB, CH), jnp.int32),
            pltpu.VMEM((2, CH, d_model), jnp.float32),
            pltpu.SemaphoreType.DMA((2,)),
            pltpu.SemaphoreType.DMA((2,)),
        ],
    )(s1, s2, s3, e1, e2, e3)


def _tc_scaled_support_body(x_ref, w_ref, wb_ref, s1_ref, s2_ref, s3_ref):
    sup = jnp.dot(x_ref[...], w_ref[...], preferred_element_type=jnp.float32)
    s1_ref[...] = wb_ref[0, 0] * sup
    s2_ref[...] = wb_ref[1, 0] * sup
    s3_ref[...] = wb_ref[2, 0] * sup


def _tc_scaled_support(x, w, wb, bm):
    n, d_model = x.shape
    grid = (n // bm,)
    blk = pl.BlockSpec((bm, d_model), lambda i: (i, 0))
    return pl.pallas_call(
        _tc_scaled_support_body,
        grid=grid,
        in_specs=[blk,
                  pl.BlockSpec((d_model, d_model), lambda i: (0, 0)),
                  pl.BlockSpec(memory_space=pltpu.SMEM)],
        out_specs=[blk, blk, blk],
        out_shape=[jax.ShapeDtypeStruct((n, d_model), jnp.float32)] * 3,
    )(x, w, wb)


def _tc_merge_support_body(p_ref, b_ref, w_ref, wb_ref,
                           u_ref, s1_ref, s2_ref, s3_ref):
    u = p_ref[0] + p_ref[1] + b_ref[...]
    u_ref[...] = u
    sup = jnp.dot(u, w_ref[...], preferred_element_type=jnp.float32)
    s1_ref[...] = wb_ref[0, 0] * sup
    s2_ref[...] = wb_ref[1, 0] * sup
    s3_ref[...] = wb_ref[2, 0] * sup


def _tc_merge_support(p, b, w, wb, bm):
    _, n, d_model = p.shape
    grid = (n // bm,)
    blk = pl.BlockSpec((bm, d_model), lambda i: (i, 0))
    return pl.pallas_call(
        _tc_merge_support_body,
        grid=grid,
        in_specs=[pl.BlockSpec((2, bm, d_model), lambda i: (0, i, 0)),
                  pl.BlockSpec((1, d_model), lambda i: (0, 0)),
                  pl.BlockSpec((d_model, d_model), lambda i: (0, 0)),
                  pl.BlockSpec(memory_space=pltpu.SMEM)],
        out_specs=[blk, blk, blk, blk],
        out_shape=[jax.ShapeDtypeStruct((n, d_model), jnp.float32)] * 4,
    )(p, b.reshape(1, d_model), w, wb)


def _tc_final_body(u1_ref, q_ref, b_ref, out_ref):
    out_ref[...] = 0.5 * (u1_ref[...] + q_ref[0] + q_ref[1] + b_ref[...])


def _tc_final(u1, q, b, bm, n):
    _, d_model = u1.shape
    grid = (n // bm,)
    blk = pl.BlockSpec((bm, d_model), lambda i: (i, 0))
    return pl.pallas_call(
        _tc_final_body,
        grid=grid,
        in_specs=[blk,
                  pl.BlockSpec((2, bm, d_model), lambda i: (0, i, 0)),
                  pl.BlockSpec((1, d_model), lambda i: (0, 0))],
        out_specs=blk,
        out_shape=jax.ShapeDtypeStruct((n, d_model), jnp.float32),
    )(u1, q, b.reshape(1, d_model))


def _pad_edges(e, n, npad, nchunk_pad):
    e = e.astype(jnp.int32)
    pad = nchunk_pad * CH - e.shape[1]
    fill = n + (jnp.arange(pad, dtype=jnp.int32) % (npad - n))
    ep = jnp.concatenate(
        [e, jnp.broadcast_to(fill, (2, pad))], axis=1)
    return ep.reshape(2, nchunk_pad // SLAB, SLAB, CH)


def kernel(x, edge_index1, edge_index2, edge_index3, weight_b, W1, b1, W2, b2):
    n, d_model = x.shape
    npad = ((n + NS * CH - 1) // (NS * CH)) * (NS * CH)
    e = edge_index1.shape[1]
    echunk = NC * NS * SLAB * CH
    nchunk_pad = ((e + echunk - 1) // echunk) * echunk // CH
    e1 = _pad_edges(edge_index1, n, npad, nchunk_pad)
    e2 = _pad_edges(edge_index2, n, npad, nchunk_pad)
    e3 = _pad_edges(edge_index3, n, npad, nchunk_pad)
    x_pad = jnp.pad(x, ((0, npad - n), (0, 0)))

    s1, s2, s3 = _tc_scaled_support(x_pad, W1, weight_b, npad // 16)
    p = _sc_spmm(s1, s2, s3, e1, e2, e3)
    u1, t1, t2, t3 = _tc_merge_support(p, b1, W2, weight_b, npad // 16)
    q = _sc_spmm(t1, t2, t3, e1, e2, e3)
    return _tc_final(u1, q, b2, n // 10, n)


# speedup vs baseline: 4.0556x; 1.0058x over previous
import functools

import jax
import jax.numpy as jnp
from jax import lax
from jax.experimental import pallas as pl
from jax.experimental.pallas import tpu as pltpu
from jax.experimental.pallas import tpu_sc as plsc

NC = 2
NS = 16
L = 16
CH = 128
SLAB = 8


def _spmm_body(npad, nrnd, s1, s2, s3, e1, e2, e3, part,
               acc, gi, rows, gsem):
    c = lax.axis_index("c")
    s = lax.axis_index("s")
    w = c * NS + s
    rpt = npad // NS

    @pl.loop(0, CH)
    def _zero_rows(i):
        for j in range(rows.shape[2] // L):
            rows[0, i, j * L:(j + 1) * L] = jnp.zeros((L,), jnp.float32)

    for k in range(rpt // CH):
        pltpu.sync_copy(rows.at[0], acc.at[pl.ds(s * rpt + k * CH, CH)])
    plsc.subcore_barrier()

    streams = ((e1, s1), (e2, s2), (e3, s3))

    nslabt = nrnd // SLAB
    assert nrnd % SLAB == 0

    for e_ref, s_ref in streams:
        base = w * nslabt

        pltpu.sync_copy(e_ref.at[0, pl.ds(base, nslabt)], gi.at[0])
        pltpu.sync_copy(e_ref.at[1, pl.ds(base, nslabt)], gi.at[1])
        pltpu.async_copy(s_ref.at[gi.at[0, 0, 0]], rows.at[0], gsem.at[0])

        @pl.loop(0, nslabt)
        def _slab(m, _s=s_ref):
            for u in range(2 * SLAB):
                b, d = divmod(u, 2)
                rb = u % 2
                pltpu.make_async_copy(_s.at[gi.at[d, m, b]], rows.at[rb],
                                      gsem.at[rb]).wait()
                if u + 1 < 2 * SLAB:
                    b2, d2 = divmod(u + 1, 2)
                    pltpu.async_copy(_s.at[gi.at[d2, m, b2]],
                                     rows.at[1 - rb], gsem.at[1 - rb])
                else:
                    @pl.when(m + 1 < nslabt)
                    def _():
                        pltpu.async_copy(_s.at[gi.at[0, m + 1, 0]],
                                         rows.at[1 - rb], gsem.at[1 - rb])
                pltpu.sync_copy(rows.at[rb], acc.at[gi.at[1 - d, m, b]],
                                add=True)

    plsc.subcore_barrier()
    pltpu.sync_copy(acc.at[pl.ds(s * rpt, rpt)],
                    part.at[c, pl.ds(s * rpt, rpt)])


def _sc_spmm(s1, s2, s3, e1, e2, e3):
    npad, d_model = s1.shape
    nchunk = e1.shape[1] * SLAB
    nrnd = nchunk // (NC * NS)
    mesh = plsc.VectorSubcoreMesh(core_axis_name="c", subcore_axis_name="s")
    body = functools.partial(_spmm_body, npad, nrnd)
    return pl.kernel(
        body,
        out_type=jax.ShapeDtypeStruct((NC, npad, d_model), jnp.float32),
        mesh=mesh,
        scratch_types=[
            pltpu.VMEM_SHARED((npad, d_model), jnp.float32),
            pltpu.VMEM((2, nrnd // SLAB, SLAB, CH), jnp.int32),
            pltpu.VMEM((2, CH, d_model), jnp.float32),
            pltpu.SemaphoreType.DMA((2,)),
        ],
    )(s1, s2, s3, e1, e2, e3)


def _tc_scaled_support_body(x_ref, w_ref, wb_ref, s1_ref, s2_ref, s3_ref):
    sup = jnp.dot(x_ref[...], w_ref[...], preferred_element_type=jnp.float32)
    s1_ref[...] = wb_ref[0, 0] * sup
    s2_ref[...] = wb_ref[1, 0] * sup
    s3_ref[...] = wb_ref[2, 0] * sup


def _tc_scaled_support(x, w, wb, bm):
    n, d_model = x.shape
    grid = (n // bm,)
    blk = pl.BlockSpec((bm, d_model), lambda i: (i, 0))
    return pl.pallas_call(
        _tc_scaled_support_body,
        grid=grid,
        in_specs=[blk,
                  pl.BlockSpec((d_model, d_model), lambda i: (0, 0)),
                  pl.BlockSpec(memory_space=pltpu.SMEM)],
        out_specs=[blk, blk, blk],
        out_shape=[jax.ShapeDtypeStruct((n, d_model), jnp.float32)] * 3,
    )(x, w, wb)


def _tc_merge_support_body(p_ref, b_ref, w_ref, wb_ref,
                           u_ref, s1_ref, s2_ref, s3_ref):
    u = p_ref[0] + p_ref[1] + b_ref[...]
    u_ref[...] = u
    sup = jnp.dot(u, w_ref[...], preferred_element_type=jnp.float32)
    s1_ref[...] = wb_ref[0, 0] * sup
    s2_ref[...] = wb_ref[1, 0] * sup
    s3_ref[...] = wb_ref[2, 0] * sup


def _tc_merge_support(p, b, w, wb, bm):
    _, n, d_model = p.shape
    grid = (n // bm,)
    blk = pl.BlockSpec((bm, d_model), lambda i: (i, 0))
    return pl.pallas_call(
        _tc_merge_support_body,
        grid=grid,
        in_specs=[pl.BlockSpec((2, bm, d_model), lambda i: (0, i, 0)),
                  pl.BlockSpec((1, d_model), lambda i: (0, 0)),
                  pl.BlockSpec((d_model, d_model), lambda i: (0, 0)),
                  pl.BlockSpec(memory_space=pltpu.SMEM)],
        out_specs=[blk, blk, blk, blk],
        out_shape=[jax.ShapeDtypeStruct((n, d_model), jnp.float32)] * 4,
    )(p, b.reshape(1, d_model), w, wb)


def _tc_final_body(u1_ref, q_ref, b_ref, out_ref):
    out_ref[...] = 0.5 * (u1_ref[...] + q_ref[0] + q_ref[1] + b_ref[...])


def _tc_final(u1, q, b, bm, n):
    _, d_model = u1.shape
    grid = (n // bm,)
    blk = pl.BlockSpec((bm, d_model), lambda i: (i, 0))
    return pl.pallas_call(
        _tc_final_body,
        grid=grid,
        in_specs=[blk,
                  pl.BlockSpec((2, bm, d_model), lambda i: (0, i, 0)),
                  pl.BlockSpec((1, d_model), lambda i: (0, 0))],
        out_specs=blk,
        out_shape=jax.ShapeDtypeStruct((n, d_model), jnp.float32),
    )(u1, q, b.reshape(1, d_model))


def _pad_edges(e, n, npad, nchunk_pad):
    e = e.astype(jnp.int32)
    pad = nchunk_pad * CH - e.shape[1]
    fill = n + (jnp.arange(pad, dtype=jnp.int32) % (npad - n))
    ep = jnp.concatenate(
        [e, jnp.broadcast_to(fill, (2, pad))], axis=1)
    return ep.reshape(2, nchunk_pad // SLAB, SLAB, CH)


def kernel(x, edge_index1, edge_index2, edge_index3, weight_b, W1, b1, W2, b2):
    n, d_model = x.shape
    npad = ((n + NS * CH - 1) // (NS * CH)) * (NS * CH)
    e = edge_index1.shape[1]
    echunk = NC * NS * SLAB * CH
    nchunk_pad = ((e + echunk - 1) // echunk) * echunk // CH
    e1 = _pad_edges(edge_index1, n, npad, nchunk_pad)
    e2 = _pad_edges(edge_index2, n, npad, nchunk_pad)
    e3 = _pad_edges(edge_index3, n, npad, nchunk_pad)
    x_pad = jnp.pad(x, ((0, npad - n), (0, 0)))

    s1, s2, s3 = _tc_scaled_support(x_pad, W1, weight_b, npad // 16)
    p = _sc_spmm(s1, s2, s3, e1, e2, e3)
    u1, t1, t2, t3 = _tc_merge_support(p, b1, W2, weight_b, npad // 16)
    q = _sc_spmm(t1, t2, t3, e1, e2, e3)
    return _tc_final(u1, q, b2, n // 10, n)
